# baseline (device time: 39699 ns/iter reference)
import jax
import jax.numpy as jnp
from jax import lax
from jax.experimental import pallas as pl
from jax.experimental.pallas import tpu as pltpu

N_DEV = 4
N_LAYERS = 3
RS, AG = 0, 1


def kernel(x, Win0, Wout0, Win1, Wout1, Win2, Wout2):
    B, D = x.shape
    R = B // N_DEV

    def body(x_ref, win0_ref, wout0_ref, win1_ref, wout1_ref, win2_ref,
             wout2_ref, out_ref, part_ref, rs_ref, ag_ref,
             send_sems, recv_sems):
        my = lax.axis_index("i")
        wins = [win0_ref, win1_ref, win2_ref]
        wouts = [wout0_ref, wout1_ref, wout2_ref]

        sent_rs = {}
        sent_ag = {}

        def remote(src, dst, l, phase, send_slot, recv_slot, target):
            return pltpu.make_async_remote_copy(
                src_ref=src, dst_ref=dst,
                send_sem=send_sems.at[l, phase, send_slot],
                recv_sem=recv_sems.at[l, phase, recv_slot],
                device_id=(target,),
                device_id_type=pl.DeviceIdType.MESH,
            )

        def mlp_chunk(xc, l):
            h = jnp.maximum(
                jnp.dot(xc, wins[l][:, :], preferred_element_type=jnp.float32),
                0.0)
            return jnp.dot(h, wouts[l][:, :],
                           preferred_element_type=jnp.float32)

        def send_chunk(val, l, o):
            e = (my + o) % N_DEV
            if l > 0:
                sent_rs[(l - 1, o)].wait_send()
            part_ref[o, :, :] = val
            rdma = remote(part_ref.at[o], rs_ref.at[my], l, RS, e, my, e)
            rdma.start()
            sent_rs[(l, o)] = rdma

        def wait_rs(l, o):
            s = (my + o) % N_DEV
            remote(part_ref.at[0], rs_ref.at[s], l, RS, s, s, s).wait_recv()
            return rs_ref[s, :, :]

        def wait_ag(l, o):
            c = (my + o) % N_DEV
            remote(ag_ref.at[c], ag_ref.at[c], l, AG, c, c, c).wait_recv()
            return ag_ref[c, :, :]

        for o in (2, 1, 3):
            c = (my + o) % N_DEV
            send_chunk(mlp_chunk(x_ref[pl.ds(c * R, R), :], 0), 0, o)
        own = mlp_chunk(x_ref[pl.ds(my * R, R), :], 0)

        for l in range(1, N_LAYERS + 1):
            total = own
            for o in (1, 3, 2):
                total = total + wait_rs(l - 1, o)

            if l == N_LAYERS:
                out_ref[:, :] = total
                break

            if l > 1:
                for o in (2, 1, 3):
                    sent_ag[(l - 2, o)].wait_send()
            ag_ref[my, :, :] = total
            for o in (2, 1, 3):
                e = (my + o) % N_DEV
                rdma = remote(ag_ref.at[my], ag_ref.at[my], l - 1, AG, e, my, e)
                rdma.start()
                sent_ag[(l - 1, o)] = rdma

            own = mlp_chunk(total, l)
            for o in (1, 3, 2):
                send_chunk(mlp_chunk(wait_ag(l - 1, o), l), l, o)

        for o in (2, 1, 3):
            sent_rs[(N_LAYERS - 1, o)].wait_send()
            sent_ag[(N_LAYERS - 2, o)].wait_send()

    return pl.pallas_call(
        body,
        out_shape=jax.ShapeDtypeStruct((R, D), jnp.float32),
        in_specs=[pl.BlockSpec(memory_space=pltpu.VMEM)] * 7,
        out_specs=pl.BlockSpec(memory_space=pltpu.VMEM),
        scratch_shapes=[
            pltpu.VMEM((N_DEV, R, D), jnp.float32),
            pltpu.VMEM((N_DEV, R, D), jnp.float32),
            pltpu.VMEM((N_DEV, R, D), jnp.float32),
            pltpu.SemaphoreType.DMA((N_LAYERS, 2, N_DEV)),
            pltpu.SemaphoreType.DMA((N_LAYERS, 2, N_DEV)),
        ],
    )(x, Win0, Wout0, Win1, Wout1, Win2, Wout2)


# device time: 32969 ns/iter; 1.2041x vs baseline; 1.2041x over previous
import jax
import jax.numpy as jnp
from jax import lax
from jax.experimental import pallas as pl
from jax.experimental.pallas import tpu as pltpu

N_DEV = 4
N_LAYERS = 3
RS, AG = 0, 1


def kernel(x, Win0, Wout0, Win1, Wout1, Win2, Wout2):
    B, D = x.shape
    H = Win0.shape[1]
    R = B // N_DEV

    def body(x_ref, win0_ref, wout0_ref, win1_ref, wout1_ref, win2_ref,
             wout2_ref, out_ref, win_b, wout_b, part_ref, rs_ref, ag_ref,
             send_sems, recv_sems):
        my = lax.axis_index("i")

        for l, (wi, wo) in enumerate([(win0_ref, wout0_ref),
                                      (win1_ref, wout1_ref),
                                      (win2_ref, wout2_ref)]):
            win_b[l, :, :] = wi[:, :].astype(jnp.bfloat16)
            wout_b[l, :, :] = wo[:, :].astype(jnp.bfloat16)

        sent_rs = {}
        sent_ag = {}

        def remote(src, dst, l, phase, send_slot, recv_slot, target):
            return pltpu.make_async_remote_copy(
                src_ref=src, dst_ref=dst,
                send_sem=send_sems.at[l, phase, send_slot],
                recv_sem=recv_sems.at[l, phase, recv_slot],
                device_id=(target,),
                device_id_type=pl.DeviceIdType.MESH,
            )

        def mlp_chunk(xc, l):
            h = jnp.maximum(
                jnp.dot(xc, win_b[l, :, :],
                        preferred_element_type=jnp.float32),
                0.0).astype(jnp.bfloat16)
            return jnp.dot(h, wout_b[l, :, :],
                           preferred_element_type=jnp.float32)

        def send_chunk(val, l, o):
            e = (my + o) % N_DEV
            if l > 0:
                sent_rs[(l - 1, o)].wait_send()
            part_ref[o, :, :] = val.astype(jnp.bfloat16)
            rdma = remote(part_ref.at[o], rs_ref.at[my], l, RS, e, my, e)
            rdma.start()
            sent_rs[(l, o)] = rdma

        def wait_rs(l, o):
            s = (my + o) % N_DEV
            remote(part_ref.at[0], rs_ref.at[s], l, RS, s, s, s).wait_recv()
            return rs_ref[s, :, :].astype(jnp.float32)

        def wait_ag(l, o):
            c = (my + o) % N_DEV
            remote(ag_ref.at[c], ag_ref.at[c], l, AG, c, c, c).wait_recv()
            return ag_ref[c, :, :]

        for o in (2, 1, 3):
            c = (my + o) % N_DEV
            xc = x_ref[pl.ds(c * R, R), :].astype(jnp.bfloat16)
            send_chunk(mlp_chunk(xc, 0), 0, o)
        own = mlp_chunk(x_ref[pl.ds(my * R, R), :].astype(jnp.bfloat16), 0)

        for l in range(1, N_LAYERS + 1):
            total = own
            for o in (1, 3, 2):
                total = total + wait_rs(l - 1, o)

            if l == N_LAYERS:
                out_ref[:, :] = total
                break

            if l > 1:
                for o in (2, 1, 3):
                    sent_ag[(l - 2, o)].wait_send()
            total_b = total.astype(jnp.bfloat16)
            ag_ref[my, :, :] = total_b
            for o in (2, 1, 3):
                e = (my + o) % N_DEV
                rdma = remote(ag_ref.at[my], ag_ref.at[my], l - 1, AG, e, my, e)
                rdma.start()
                sent_ag[(l - 1, o)] = rdma

            own = mlp_chunk(total_b, l)
            for o in (1, 3, 2):
                send_chunk(mlp_chunk(wait_ag(l - 1, o), l), l, o)

        for o in (2, 1, 3):
            sent_rs[(N_LAYERS - 1, o)].wait_send()
            sent_ag[(N_LAYERS - 2, o)].wait_send()

    return pl.pallas_call(
        body,
        out_shape=jax.ShapeDtypeStruct((R, D), jnp.float32),
        in_specs=[pl.BlockSpec(memory_space=pltpu.VMEM)] * 7,
        out_specs=pl.BlockSpec(memory_space=pltpu.VMEM),
        scratch_shapes=[
            pltpu.VMEM((N_LAYERS, D, H), jnp.bfloat16),
            pltpu.VMEM((N_LAYERS, H, D), jnp.bfloat16),
            pltpu.VMEM((N_DEV, R, D), jnp.bfloat16),
            pltpu.VMEM((N_DEV, R, D), jnp.bfloat16),
            pltpu.VMEM((N_DEV, R, D), jnp.bfloat16),
            pltpu.SemaphoreType.DMA((N_LAYERS, 2, N_DEV)),
            pltpu.SemaphoreType.DMA((N_LAYERS, 2, N_DEV)),
        ],
    )(x, Win0, Wout0, Win1, Wout1, Win2, Wout2)


# device time: 29772 ns/iter; 1.3334x vs baseline; 1.1074x over previous
import jax
import jax.numpy as jnp
from jax import lax
from jax.experimental import pallas as pl
from jax.experimental.pallas import tpu as pltpu

N_DEV = 4
N_LAYERS = 3


def kernel(x, Win0, Wout0, Win1, Wout1, Win2, Wout2):
    B, D = x.shape
    H = Win0.shape[1]
    R = B // N_DEV

    def body(x_ref, win0_ref, wout0_ref, win1_ref, wout1_ref, win2_ref,
             wout2_ref, out_ref, win_b, wout_b, bc_ref, part_ref, rs_ref,
             send_b, recv_b, send_rs, recv_rs):
        my = lax.axis_index("i")

        for l, (wi, wo) in enumerate([(win0_ref, wout0_ref),
                                      (win1_ref, wout1_ref),
                                      (win2_ref, wout2_ref)]):
            win_b[l, :, :] = wi[:, :].astype(jnp.bfloat16)
            wout_b[l, :, :] = wo[:, :].astype(jnp.bfloat16)

        started = []

        def mlp_chunk(xc, l):
            h = jnp.maximum(
                jnp.dot(xc, win_b[l, :, :],
                        preferred_element_type=jnp.float32),
                0.0).astype(jnp.bfloat16)
            return jnp.dot(h, wout_b[l, :, :],
                           preferred_element_type=jnp.float32)

        def bcast_chunk(l, c):
            src = bc_ref.at[l, my, pl.ds(c * R, R), :]
            for o in (2, 1, 3):
                e = (my + o) % N_DEV
                rdma = pltpu.make_async_remote_copy(
                    src_ref=src, dst_ref=src,
                    send_sem=send_b.at[l, c, o - 1],
                    recv_sem=recv_b.at[l, my, c],
                    device_id=(e,), device_id_type=pl.DeviceIdType.MESH,
                )
                rdma.start()
                started.append(rdma)

        def gather_chunk(l, c):
            acc = bc_ref[l, my, pl.ds(c * R, R), :].astype(jnp.float32)
            for o in (1, 3, 2):
                s = (my + o) % N_DEV
                pltpu.make_async_remote_copy(
                    src_ref=bc_ref.at[l, s, pl.ds(c * R, R), :],
                    dst_ref=bc_ref.at[l, s, pl.ds(c * R, R), :],
                    send_sem=send_b.at[l, c, 0],
                    recv_sem=recv_b.at[l, s, c],
                    device_id=(s,), device_id_type=pl.DeviceIdType.MESH,
                ).wait_recv()
                acc = acc + bc_ref[l, s, pl.ds(c * R, R), :].astype(jnp.float32)
            return acc

        def rs_send(c):
            return pltpu.make_async_remote_copy(
                src_ref=part_ref.at[c],
                dst_ref=rs_ref.at[my],
                send_sem=send_rs.at[c],
                recv_sem=recv_rs.at[my],
                device_id=(c,), device_id_type=pl.DeviceIdType.MESH,
            )

        for l in (0, 1):
            for c in range(N_DEV):
                if l == 0:
                    xc = x_ref[pl.ds(c * R, R), :].astype(jnp.bfloat16)
                else:
                    xc = gather_chunk(0, c).astype(jnp.bfloat16)
                bc_ref[l, my, pl.ds(c * R, R), :] = \
                    mlp_chunk(xc, l).astype(jnp.bfloat16)
                bcast_chunk(l, c)

        for c in range(N_DEV):
            xc = gather_chunk(1, c).astype(jnp.bfloat16)
            part_ref[c, :, :] = mlp_chunk(xc, 2).astype(jnp.bfloat16)

            @pl.when(c != my)
            def _():
                rs_send(c).start()

        acc = part_ref[my, :, :].astype(jnp.float32)
        for o in (1, 3, 2):
            s = (my + o) % N_DEV
            pltpu.make_async_remote_copy(
                src_ref=rs_ref.at[s], dst_ref=rs_ref.at[s],
                send_sem=send_rs.at[0], recv_sem=recv_rs.at[s],
                device_id=(s,), device_id_type=pl.DeviceIdType.MESH,
            ).wait_recv()
            acc = acc + rs_ref[s, :, :].astype(jnp.float32)
        out_ref[:, :] = acc

        for rdma in started:
            rdma.wait_send()
        for c in range(N_DEV):
            @pl.when(c != my)
            def _():
                rs_send(c).wait_send()

    return pl.pallas_call(
        body,
        out_shape=jax.ShapeDtypeStruct((R, D), jnp.float32),
        in_specs=[pl.BlockSpec(memory_space=pltpu.VMEM)] * 7,
        out_specs=pl.BlockSpec(memory_space=pltpu.VMEM),
        scratch_shapes=[
            pltpu.VMEM((N_LAYERS, D, H), jnp.bfloat16),
            pltpu.VMEM((N_LAYERS, H, D), jnp.bfloat16),
            pltpu.VMEM((2, N_DEV, B, D), jnp.bfloat16),
            pltpu.VMEM((N_DEV, R, D), jnp.bfloat16),
            pltpu.VMEM((N_DEV, R, D), jnp.bfloat16),
            pltpu.SemaphoreType.DMA((2, N_DEV, N_DEV - 1)),
            pltpu.SemaphoreType.DMA((2, N_DEV, N_DEV)),
            pltpu.SemaphoreType.DMA((N_DEV,)),
            pltpu.SemaphoreType.DMA((N_DEV,)),
        ],
    )(x, Win0, Wout0, Win1, Wout1, Win2, Wout2)
